# baseline (speedup 1.0000x reference)
import jax
import jax.numpy as jnp
from jax import lax
from jax.experimental import pallas as pl
from jax.experimental.pallas import tpu as pltpu
from jax.experimental.pallas import tpu_sc as plsc

B = 4
S = 2048
N_BN = 32
D = 1024
SEQ_OUT = S + N_BN
NW = 32
IDX_PER_W = (B * S) // NW
CW = 32
NCHUNK = IDX_PER_W // CW
NB = 3
S_PER_W = S // (NW // B)


def kernel(lang_x, embedding_table, bn_embedding):
    idx = lang_x.reshape(B * S)
    mesh = plsc.VectorSubcoreMesh(core_axis_name="c", subcore_axis_name="s")

    @pl.kernel(
        out_type=jax.ShapeDtypeStruct((B * SEQ_OUT, D), embedding_table.dtype),
        mesh=mesh,
        scratch_types=[
            pltpu.VMEM((IDX_PER_W,), jnp.int32),
            pltpu.VMEM((CW, D), jnp.float32),
            pltpu.VMEM((CW, D), jnp.float32),
            pltpu.VMEM((CW, D), jnp.float32),
            pltpu.VMEM((8, D), jnp.float32),
            pltpu.SemaphoreType.DMA,
            pltpu.SemaphoreType.DMA,
            pltpu.SemaphoreType.DMA,
            pltpu.SemaphoreType.DMA,
            pltpu.SemaphoreType.DMA,
            pltpu.SemaphoreType.DMA,
            pltpu.SemaphoreType.DMA,
        ],
    )
    def emb_kernel(
        table_hbm, idx_hbm, bn_hbm, out_hbm,
        idx_v, rows_a, rows_b, rows_c, bn_v,
        sem_ga, sem_gb, sem_gc, sem_oa, sem_ob, sem_oc, sem_bn,
    ):
        wid = lax.axis_index("s") * 2 + lax.axis_index("c")
        base = wid * IDX_PER_W
        batch = wid // (NW // B)
        row0 = batch * SEQ_OUT + (wid % (NW // B)) * S_PER_W

        pltpu.sync_copy(idx_hbm.at[pl.ds(base, IDX_PER_W)], idx_v)

        bufs = (rows_a, rows_b, rows_c)
        gsems = (sem_ga, sem_gb, sem_gc)
        osems = (sem_oa, sem_ob, sem_oc)

        def gath(c):
            return pltpu.async_copy(
                table_hbm.at[idx_v.at[pl.ds(c * CW, CW)]],
                bufs[c % NB],
                gsems[c % NB],
            )

        gathers = [None] * NCHUNK
        outs = [None] * NCHUNK
        for _c in range(NB - 1):
            gathers[_c] = gath(_c)

        @pl.when(wid < 16)
        def _():
            j = wid % 4
            pltpu.async_copy(bn_hbm.at[pl.ds(j * 8, 8)], bn_v, sem_bn)

        LA = NB - 1
        for c in range(NCHUNK):
            if c + LA < NCHUNK:
                if c >= 1:
                    outs[c - 1].wait()
                gathers[c + LA] = gath(c + LA)
            gathers[c].wait()
            outs[c] = pltpu.async_copy(
                bufs[c % NB], out_hbm.at[pl.ds(row0 + c * CW, CW)], osems[c % NB]
            )
            if c == 1:
                @pl.when(wid < 16)
                def _():
                    b = wid // 4
                    j = wid % 4
                    pltpu.make_async_copy(
                        bn_hbm.at[pl.ds(j * 8, 8)], bn_v, sem_bn
                    ).wait()
                    pltpu.async_copy(
                        bn_v,
                        out_hbm.at[pl.ds(b * SEQ_OUT + S + j * 8, 8)],
                        sem_bn,
                    )
        for _c in range(max(0, NCHUNK - NB), NCHUNK):
            outs[_c].wait()

        @pl.when(wid < 16)
        def _():
            b = wid // 4
            j = wid % 4
            pltpu.make_async_copy(
                bn_v, out_hbm.at[pl.ds(b * SEQ_OUT + S + j * 8, 8)], sem_bn
            ).wait()

    out = emb_kernel(embedding_table, idx, bn_embedding)
    return out.reshape(B, SEQ_OUT, D)


# speedup vs baseline: 1.0209x; 1.0209x over previous
import jax
import jax.numpy as jnp
from jax import lax
from jax.experimental import pallas as pl
from jax.experimental.pallas import tpu as pltpu
from jax.experimental.pallas import tpu_sc as plsc

B = 4
S = 2048
N_BN = 32
D = 1024
SEQ_OUT = S + N_BN
NW = 32
IDX_PER_W = (B * S) // NW
CW = 32
NCHUNK = IDX_PER_W // CW
NB = 3
S_PER_W = S // (NW // B)


def kernel(lang_x, embedding_table, bn_embedding):
    mesh = plsc.VectorSubcoreMesh(core_axis_name="c", subcore_axis_name="s")

    @pl.kernel(
        out_type=jax.ShapeDtypeStruct((B * SEQ_OUT, D), embedding_table.dtype),
        mesh=mesh,
        scratch_types=[
            pltpu.VMEM((IDX_PER_W,), jnp.int32),
            pltpu.VMEM((CW, D), jnp.float32),
            pltpu.VMEM((CW, D), jnp.float32),
            pltpu.VMEM((CW, D), jnp.float32),
            pltpu.VMEM((8, D), jnp.float32),
            pltpu.SemaphoreType.DMA,
            pltpu.SemaphoreType.DMA,
            pltpu.SemaphoreType.DMA,
            pltpu.SemaphoreType.DMA,
            pltpu.SemaphoreType.DMA,
            pltpu.SemaphoreType.DMA,
            pltpu.SemaphoreType.DMA,
        ],
    )
    def emb_kernel(
        table_hbm, idx_hbm, bn_hbm, out_hbm,
        idx_v, rows_a, rows_b, rows_c, bn_v,
        sem_ga, sem_gb, sem_gc, sem_oa, sem_ob, sem_oc, sem_bn,
    ):
        wid = lax.axis_index("s") * 2 + lax.axis_index("c")
        batch = wid // (NW // B)
        quarter = wid % (NW // B)
        row0 = batch * SEQ_OUT + quarter * S_PER_W

        pltpu.sync_copy(
            idx_hbm.at[batch, pl.ds(quarter * IDX_PER_W, IDX_PER_W)], idx_v
        )

        bufs = (rows_a, rows_b, rows_c)
        gsems = (sem_ga, sem_gb, sem_gc)
        osems = (sem_oa, sem_ob, sem_oc)

        def gath(c):
            return pltpu.async_copy(
                table_hbm.at[idx_v.at[pl.ds(c * CW, CW)]],
                bufs[c % NB],
                gsems[c % NB],
            )

        gathers = [None] * NCHUNK
        outs = [None] * NCHUNK
        for _c in range(NB - 1):
            gathers[_c] = gath(_c)

        @pl.when(wid < 16)
        def _():
            b = wid // 4
            j = wid % 4
            pltpu.async_copy(bn_hbm.at[pl.ds(j * 8, 8)], bn_v, sem_bn).wait()
            pltpu.async_copy(
                bn_v, out_hbm.at[pl.ds(b * SEQ_OUT + S + j * 8, 8)], sem_bn
            ).wait()

        LA = NB - 1
        for c in range(NCHUNK):
            if c + LA < NCHUNK:
                if c >= 1:
                    outs[c - 1].wait()
                gathers[c + LA] = gath(c + LA)
            gathers[c].wait()
            outs[c] = pltpu.async_copy(
                bufs[c % NB], out_hbm.at[pl.ds(row0 + c * CW, CW)], osems[c % NB]
            )
        for _c in range(max(0, NCHUNK - NB), NCHUNK):
            outs[_c].wait()

    out = emb_kernel(embedding_table, lang_x, bn_embedding)
    return out.reshape(B, SEQ_OUT, D)


# speedup vs baseline: 1.0210x; 1.0001x over previous
import jax
import jax.numpy as jnp
from jax import lax
from jax.experimental import pallas as pl
from jax.experimental.pallas import tpu as pltpu
from jax.experimental.pallas import tpu_sc as plsc

B = 4
S = 2048
N_BN = 32
D = 1024
SEQ_OUT = S + N_BN
NW = 32
IDX_PER_W = (B * S) // NW
CW = 32
NCHUNK = IDX_PER_W // CW
NB = 3
S_PER_W = S // (NW // B)


def kernel(lang_x, embedding_table, bn_embedding):
    mesh = plsc.VectorSubcoreMesh(core_axis_name="c", subcore_axis_name="s")

    @pl.kernel(
        out_type=jax.ShapeDtypeStruct((B * SEQ_OUT, D), embedding_table.dtype),
        mesh=mesh,
        scratch_types=[
            pltpu.VMEM((IDX_PER_W,), jnp.int32),
            pltpu.VMEM((CW, D), jnp.float32),
            pltpu.VMEM((CW, D), jnp.float32),
            pltpu.VMEM((CW, D), jnp.float32),
            pltpu.VMEM((8, D), jnp.float32),
            pltpu.SemaphoreType.DMA,
            pltpu.SemaphoreType.DMA,
            pltpu.SemaphoreType.DMA,
            pltpu.SemaphoreType.DMA,
            pltpu.SemaphoreType.DMA,
            pltpu.SemaphoreType.DMA,
            pltpu.SemaphoreType.DMA,
            pltpu.SemaphoreType.DMA,
            pltpu.SemaphoreType.DMA,
            pltpu.SemaphoreType.DMA,
        ],
    )
    def emb_kernel(
        table_hbm, idx_hbm, bn_hbm, out_hbm,
        idx_v, rows_a, rows_b, rows_c, bn_v,
        sem_ga, sem_gb, sem_gc, sem_oa, sem_ob, sem_oc,
        sem_oa2, sem_ob2, sem_oc2, sem_bn,
    ):
        wid = lax.axis_index("s") * 2 + lax.axis_index("c")
        batch = wid // (NW // B)
        quarter = wid % (NW // B)
        row0 = batch * SEQ_OUT + quarter * S_PER_W

        pltpu.sync_copy(
            idx_hbm.at[batch, pl.ds(quarter * IDX_PER_W, IDX_PER_W)], idx_v
        )

        bufs = (rows_a, rows_b, rows_c)
        gsems = (sem_ga, sem_gb, sem_gc)
        osems = (sem_oa, sem_ob, sem_oc)
        osems2 = (sem_oa2, sem_ob2, sem_oc2)

        def gath(c):
            return pltpu.async_copy(
                table_hbm.at[idx_v.at[pl.ds(c * CW, CW)]],
                bufs[c % NB],
                gsems[c % NB],
            )

        gathers = [None] * NCHUNK
        outs = [None] * NCHUNK
        for _c in range(NB - 1):
            gathers[_c] = gath(_c)

        @pl.when(wid < 16)
        def _():
            b = wid // 4
            j = wid % 4
            pltpu.async_copy(bn_hbm.at[pl.ds(j * 8, 8)], bn_v, sem_bn).wait()
            pltpu.async_copy(
                bn_v, out_hbm.at[pl.ds(b * SEQ_OUT + S + j * 8, 8)], sem_bn
            ).wait()

        LA = NB - 1
        for c in range(NCHUNK):
            if c + LA < NCHUNK:
                if c >= 1:
                    outs[c - 1][0].wait()
                    outs[c - 1][1].wait()
                gathers[c + LA] = gath(c + LA)
            gathers[c].wait()
            H = CW // 2
            outs[c] = (
                pltpu.async_copy(
                    bufs[c % NB].at[pl.ds(0, H)],
                    out_hbm.at[pl.ds(row0 + c * CW, H)],
                    osems[c % NB],
                ),
                pltpu.async_copy(
                    bufs[c % NB].at[pl.ds(H, H)],
                    out_hbm.at[pl.ds(row0 + c * CW + H, H)],
                    osems2[c % NB],
                ),
            )
        for _c in range(max(0, NCHUNK - NB), NCHUNK):
            outs[_c][0].wait()
            outs[_c][1].wait()

    out = emb_kernel(embedding_table, lang_x, bn_embedding)
    return out.reshape(B, SEQ_OUT, D)
